# baseline (speedup 1.0000x reference)
import functools

import jax
import jax.numpy as jnp
from jax import lax
from jax.experimental import pallas as pl
from jax.experimental.pallas import tpu as pltpu
from jax.experimental.pallas import tpu_sc as plsc

BATCH = 16384
N_FIXED = 128
K_DIM = 32
VOCAB = 1_000_000

_NC = 2
_NS = 16
_NW = _NC * _NS
_BPW = BATCH // _NW
_G = 8
_NG = _BPW // _G


def _make_sc_gather():
    mesh = plsc.VectorSubcoreMesh(core_axis_name="c", subcore_axis_name="s")

    @functools.partial(
        pl.kernel,
        mesh=mesh,
        out_type=jax.ShapeDtypeStruct((4, 8, BATCH), jnp.float32),
        scratch_types=[
            pltpu.VMEM((_BPW,), jnp.int32),
            pltpu.VMEM((2 * _G * 4, 8, 128), jnp.float32),
            pltpu.VMEM((4, 8, _BPW), jnp.float32),
            pltpu.SemaphoreType.DMA,
        ],
        compiler_params=pltpu.CompilerParams(needs_layout_passes=False),
    )
    def gather_kernel(table_hbm, idx_hbm, out_hbm, idx_v, stage_v, rows_v, sem):
        wid = lax.axis_index("s") * _NC + lax.axis_index("c")
        base = wid * _BPW
        pltpu.sync_copy(idx_hbm.at[pl.ds(base, _BPW)], idx_v)

        def pair(i, _):
            vec16 = idx_v[pl.ds(i * 2 * _G, 16)]
            tvec = vec16 >> 7
            lvec16 = vec16 & 127
            all_copies = []
            for h in range(2):
                for j in range(_G):
                    t128 = pl.multiple_of(tvec[h * _G + j] * 128, 128)
                    all_copies.append(
                        pltpu.async_copy(
                            table_hbm.at[:, :, pl.ds(t128, 128)],
                            stage_v.at[pl.ds(h * (_G * 4) + j * 4, 4)],
                            sem,
                        )
                    )
            for c in all_copies:
                c.wait()
            clo = lax.iota(jnp.int32, 16)
            klo, slo = clo >> 3, clo & 7
            for h in range(2):
                for j in range(_G):
                    row = h * (_G * 4) + j * 4
                    lane = clo * 0 + lvec16[h * _G + j]
                    g0 = plsc.load_gather(stage_v, [klo + row, slo, lane])
                    g1 = plsc.load_gather(stage_v, [klo + row + 2, slo, lane])
                    samp = clo * 0 + ((i * 2 + h) * _G + j)
                    plsc.store_scatter(rows_v, [klo, slo, samp], g0)
                    plsc.store_scatter(rows_v, [klo + 2, slo, samp], g1)
            return ()

        lax.fori_loop(0, _NG // 2, pair, (), unroll=False)
        pltpu.sync_copy(rows_v, out_hbm.at[:, :, pl.ds(base, _BPW)])

    return gather_kernel


def _tc_matmul_body(w_ref, x_ref, b_ref, o_ref):
    o_ref[...] = (
        jax.lax.dot_general(
            w_ref[...],
            x_ref[...],
            (((1,), (1,)), ((), ())),
            preferred_element_type=jnp.float32,
        )
        + b_ref[...]
    )


def _tc_add_body(f_ref, g_ref, o_ref):
    o_ref[...] = f_ref[...] + g_ref[...]


def kernel(X, group_ids, beta_w, beta_b, u_table):
    gids = group_ids.astype(jnp.int32)
    table3 = u_table.T.reshape(4, 8, VOCAB)
    rand3 = _make_sc_gather()(table3, gids)
    rand_t = rand3.reshape(K_DIM, BATCH)
    b2d = beta_b.reshape(K_DIM, 1)
    fixed_t = pl.pallas_call(
        _tc_matmul_body,
        out_shape=jax.ShapeDtypeStruct((K_DIM, BATCH), jnp.float32),
    )(beta_w, X, b2d)
    logits_t = pl.pallas_call(
        _tc_add_body,
        out_shape=jax.ShapeDtypeStruct((K_DIM, BATCH), jnp.float32),
    )(fixed_t, rand_t)
    return logits_t.T


# speedup vs baseline: 1.0832x; 1.0832x over previous
import functools

import jax
import jax.numpy as jnp
from jax import lax
from jax.experimental import pallas as pl
from jax.experimental.pallas import tpu as pltpu
from jax.experimental.pallas import tpu_sc as plsc

BATCH = 16384
N_FIXED = 128
K_DIM = 32
VOCAB = 1_000_000

_NC = 2
_NS = 16
_NW = _NC * _NS
_BPW = BATCH // _NW
_G = 16
_NG = _BPW // _G


def _make_sc_gather():
    mesh = plsc.VectorSubcoreMesh(core_axis_name="c", subcore_axis_name="s")

    @functools.partial(
        pl.kernel,
        mesh=mesh,
        out_type=jax.ShapeDtypeStruct((4, 8, BATCH), jnp.float32),
        scratch_types=[
            pltpu.VMEM((_BPW,), jnp.int32),
            pltpu.VMEM((_G * 4, 8, 128), jnp.float32),
            pltpu.VMEM((4, 8, _BPW), jnp.float32),
            pltpu.SemaphoreType.DMA,
        ],
        compiler_params=pltpu.CompilerParams(needs_layout_passes=False),
    )
    def gather_kernel(table_hbm, idx_hbm, out_hbm, idx_v, stage_v, rows_v, sem):
        wid = lax.axis_index("s") * _NC + lax.axis_index("c")
        base = wid * _BPW
        pltpu.sync_copy(idx_hbm.at[pl.ds(base, _BPW)], idx_v)

        def group(i, _):
            vec = idx_v[pl.ds(i * _G, _G)]
            tvec = vec >> 7
            lvec = vec & 127
            copies = []
            for j in range(_G):
                t128 = pl.multiple_of(tvec[j] * 128, 128)
                copies.append(
                    pltpu.async_copy(
                        table_hbm.at[:, :, pl.ds(t128, 128)],
                        stage_v.at[pl.ds(j * 4, 4)],
                        sem,
                    )
                )
            clo = lax.iota(jnp.int32, 16)
            klo, slo = clo >> 3, clo & 7
            for j in range(_G):
                copies[j].wait()
                lane = clo * 0 + lvec[j]
                g0 = plsc.load_gather(stage_v, [klo + j * 4, slo, lane])
                g1 = plsc.load_gather(stage_v, [klo + 2 + j * 4, slo, lane])
                samp = clo * 0 + (i * _G + j)
                plsc.store_scatter(rows_v, [klo, slo, samp], g0)
                plsc.store_scatter(rows_v, [klo + 2, slo, samp], g1)
            return ()

        lax.fori_loop(0, _NG, group, (), unroll=False)
        pltpu.sync_copy(rows_v, out_hbm.at[:, :, pl.ds(base, _BPW)])

    return gather_kernel


def _tc_matmul_body(w_ref, x_ref, b_ref, o_ref):
    o_ref[...] = (
        jax.lax.dot_general(
            w_ref[...],
            x_ref[...],
            (((1,), (1,)), ((), ())),
            preferred_element_type=jnp.float32,
        )
        + b_ref[...]
    )


def _tc_add_body(f_ref, g_ref, o_ref):
    o_ref[...] = f_ref[...] + g_ref[...]


def kernel(X, group_ids, beta_w, beta_b, u_table):
    gids = group_ids.astype(jnp.int32)
    table3 = u_table.T.reshape(4, 8, VOCAB)
    rand3 = _make_sc_gather()(table3, gids)
    rand_t = rand3.reshape(K_DIM, BATCH)
    b2d = beta_b.reshape(K_DIM, 1)
    fixed_t = pl.pallas_call(
        _tc_matmul_body,
        out_shape=jax.ShapeDtypeStruct((K_DIM, BATCH), jnp.float32),
    )(beta_w, X, b2d)
    logits_t = pl.pallas_call(
        _tc_add_body,
        out_shape=jax.ShapeDtypeStruct((K_DIM, BATCH), jnp.float32),
    )(fixed_t, rand_t)
    return logits_t.T
